# Initial kernel scaffold; baseline (speedup 1.0000x reference)
import jax
import jax.numpy as jnp
from jax.experimental import pallas as pl


def kernel(user, item, embed_user, embed_item, u_bias, i_bias, bias_):
    raise NotImplementedError("write your pallas kernel here")


# speedup vs baseline: 1.4235x; 1.4235x over previous
import functools

import jax
import jax.numpy as jnp
from jax import lax
from jax.experimental import pallas as pl
from jax.experimental.pallas import tpu as pltpu
from jax.experimental.pallas import tpu_sc as plsc

B = 256
F = 128
L = 16
R = 16


def _body(user_h, item_h, eu_h, ei_h, ub_h, ib_h, b_h, out_h,
          idx_u, idx_i, eu_v, ei_v, ub_v, ib_v, b_v, s_loc,
          s_half, out_loc, s_sh, sem):
    cid = lax.axis_index("c")
    sid = lax.axis_index("s")
    base = sid * R
    col0 = cid * F

    pltpu.sync_copy(user_h.at[pl.ds(base, R)], idx_u)
    pltpu.sync_copy(item_h.at[pl.ds(base, R)], idx_i)
    c1 = pltpu.async_copy(eu_h.at[idx_u], eu_v, sem)
    c2 = pltpu.async_copy(ei_h.at[idx_i], ei_v, sem)
    c3 = pltpu.async_copy(ub_h.at[idx_u], ub_v, sem)
    c4 = pltpu.async_copy(ib_h.at[idx_i], ib_v, sem)
    pltpu.sync_copy(b_h, b_v)
    c1.wait()
    c2.wait()
    c3.wait()
    c4.wait()

    lane = lax.iota(jnp.int32, L)
    s_vec = jnp.zeros((L,), jnp.float32)
    for r in range(R):
        acc = eu_v[r, pl.ds(0, L)] * ei_v[r, pl.ds(0, L)]
        for j in range(1, F // L):
            acc = acc + eu_v[r, pl.ds(j * L, L)] * ei_v[r, pl.ds(j * L, L)]
        s_vec = jnp.where(lane == r, jnp.sum(acc), s_vec)
    s_loc[...] = s_vec

    pltpu.sync_copy(s_loc, s_sh.at[pl.ds(base, R)])
    plsc.subcore_barrier()
    pltpu.sync_copy(s_sh.at[pl.ds(col0, F)], s_half)

    rb16 = ub_v[...] + ib_v[...] + b_v[...]

    @pl.when(cid == 0)
    def _assemble_user_half():
        for r in range(R):
            rb = rb16[r]
            for j in range(F // L):
                out_loc[r, pl.ds(j * L, L)] = (
                    eu_v[r, pl.ds(j * L, L)] + s_half[pl.ds(j * L, L)] + rb)

    @pl.when(cid == 1)
    def _assemble_item_half():
        for r in range(R):
            rb = rb16[r]
            for j in range(F // L):
                out_loc[r, pl.ds(j * L, L)] = (
                    ei_v[r, pl.ds(j * L, L)] + s_half[pl.ds(j * L, L)] + rb)

    pltpu.sync_copy(out_loc, out_h.at[pl.ds(base, R), pl.ds(col0, F)])


def kernel(user, item, embed_user, embed_item, u_bias, i_bias, bias_):
    b16 = jnp.broadcast_to(bias_, (L,)).astype(jnp.float32)
    ub = u_bias.reshape(-1)
    ib = i_bias.reshape(-1)
    mesh = plsc.VectorSubcoreMesh(core_axis_name="c", subcore_axis_name="s")
    run = functools.partial(
        pl.kernel,
        mesh=mesh,
        compiler_params=pltpu.CompilerParams(needs_layout_passes=False),
        out_type=jax.ShapeDtypeStruct((B, B), jnp.float32),
        scratch_types=[
            pltpu.VMEM((R,), jnp.int32),
            pltpu.VMEM((R,), jnp.int32),
            pltpu.VMEM((R, F), jnp.float32),
            pltpu.VMEM((R, F), jnp.float32),
            pltpu.VMEM((R,), jnp.float32),
            pltpu.VMEM((R,), jnp.float32),
            pltpu.VMEM((L,), jnp.float32),
            pltpu.VMEM((R,), jnp.float32),
            pltpu.VMEM((F,), jnp.float32),
            pltpu.VMEM((R, F), jnp.float32),
            pltpu.VMEM_SHARED((B,), jnp.float32),
            pltpu.SemaphoreType.DMA,
        ],
    )(_body)
    out2 = run(user, item, embed_user, embed_item, ub, ib, b16)
    return out2.reshape(-1)


# speedup vs baseline: 1.5037x; 1.0564x over previous
import functools

import jax
import jax.numpy as jnp
from jax import lax
from jax.experimental import pallas as pl
from jax.experimental.pallas import tpu as pltpu
from jax.experimental.pallas import tpu_sc as plsc

B = 256
F = 128
L = 16
R = 16


def _body(user_h, item_h, eu_h, ei_h, ub_h, ib_h, b_h, out_h,
          idx_u, idx_i, eu_v, ei_v, ub_v, ib_v, b_v, s_loc, acc_t,
          s_half, out_loc, s_sh, sem, gsem):
    cid = lax.axis_index("c")
    sid = lax.axis_index("s")
    base = sid * R
    col0 = cid * F

    ci = pltpu.async_copy(user_h.at[pl.ds(base, R)], idx_u, sem)
    cj = pltpu.async_copy(item_h.at[pl.ds(base, R)], idx_i, sem)
    cb = pltpu.async_copy(b_h, b_v, sem)
    ci.wait()
    cj.wait()
    c1 = pltpu.async_copy(eu_h.at[idx_u], eu_v, gsem)
    c2 = pltpu.async_copy(ei_h.at[idx_i], ei_v, gsem)
    c3 = pltpu.async_copy(ub_h.at[idx_u], ub_v, gsem)
    c4 = pltpu.async_copy(ib_h.at[idx_i], ib_v, gsem)
    c1.wait()
    c2.wait()

    lane = lax.iota(jnp.int32, L)
    for r in range(R):
        acc = eu_v[r, pl.ds(0, L)] * ei_v[r, pl.ds(0, L)]
        for j in range(1, F // L):
            acc = acc + eu_v[r, pl.ds(j * L, L)] * ei_v[r, pl.ds(j * L, L)]
        acc_t[r, :] = acc
    s_vec = plsc.load_gather(acc_t, [lane, jnp.zeros((L,), jnp.int32)])
    for j in range(1, L):
        s_vec = s_vec + plsc.load_gather(
            acc_t, [lane, jnp.full((L,), j, jnp.int32)])
    s_loc[...] = s_vec

    pltpu.sync_copy(s_loc, s_sh.at[pl.ds(base, R)])
    c3.wait()
    c4.wait()
    cb.wait()
    rb16 = ub_v[...] + ib_v[...] + b_v[...]
    plsc.subcore_barrier()
    pltpu.sync_copy(s_sh.at[pl.ds(col0, F)], s_half)

    s_reg = [s_half[pl.ds(j * L, L)] for j in range(F // L)]

    @pl.when(cid == 0)
    def _assemble_user_half():
        for r in range(R):
            rb = rb16[r]
            for j in range(F // L):
                out_loc[r, pl.ds(j * L, L)] = (
                    eu_v[r, pl.ds(j * L, L)] + (s_reg[j] + rb))

    @pl.when(cid == 1)
    def _assemble_item_half():
        for r in range(R):
            rb = rb16[r]
            for j in range(F // L):
                out_loc[r, pl.ds(j * L, L)] = (
                    ei_v[r, pl.ds(j * L, L)] + (s_reg[j] + rb))

    pltpu.sync_copy(out_loc, out_h.at[pl.ds(base, R), pl.ds(col0, F)])


def kernel(user, item, embed_user, embed_item, u_bias, i_bias, bias_):
    b16 = jnp.broadcast_to(bias_, (L,)).astype(jnp.float32)
    ub = u_bias.reshape(-1)
    ib = i_bias.reshape(-1)
    mesh = plsc.VectorSubcoreMesh(core_axis_name="c", subcore_axis_name="s")
    run = functools.partial(
        pl.kernel,
        mesh=mesh,
        compiler_params=pltpu.CompilerParams(
            needs_layout_passes=False,
            disable_bounds_checks=True,
            disable_semaphore_checks=True,
        ),
        out_type=jax.ShapeDtypeStruct((B, B), jnp.float32),
        scratch_types=[
            pltpu.VMEM((R,), jnp.int32),
            pltpu.VMEM((R,), jnp.int32),
            pltpu.VMEM((R, F), jnp.float32),
            pltpu.VMEM((R, F), jnp.float32),
            pltpu.VMEM((R,), jnp.float32),
            pltpu.VMEM((R,), jnp.float32),
            pltpu.VMEM((L,), jnp.float32),
            pltpu.VMEM((R,), jnp.float32),
            pltpu.VMEM((R, L), jnp.float32),
            pltpu.VMEM((F,), jnp.float32),
            pltpu.VMEM((R, F), jnp.float32),
            pltpu.VMEM_SHARED((B,), jnp.float32),
            pltpu.SemaphoreType.DMA,
            pltpu.SemaphoreType.DMA,
        ],
    )(_body)
    out2 = run(user, item, embed_user, embed_item, ub, ib, b16)
    return out2.reshape(-1)


# speedup vs baseline: 1.6084x; 1.0696x over previous
import functools

import jax
import jax.numpy as jnp
from jax import lax
from jax.experimental import pallas as pl
from jax.experimental.pallas import tpu as pltpu
from jax.experimental.pallas import tpu_sc as plsc

B = 256
F = 128
L = 16
R = 16
RW = 8


def _body(user_h, item_h, eu_h, ei_h, ub_h, ib_h, b_h, out_h,
          idx_u, idx_i, z_idx, eu_v, ei_v, ub_v, ib_v, b_v, s_loc, acc_t,
          s_full, out_loc, s_sh, sem, gsem):
    cid = lax.axis_index("c")
    sid = lax.axis_index("s")
    base = sid * R

    z_idx[...] = jnp.zeros((L,), jnp.int32)
    ci = pltpu.async_copy(user_h.at[pl.ds(base, R)], idx_u, sem)
    cj = pltpu.async_copy(item_h.at[pl.ds(base, R)], idx_i, sem)
    ci.wait()
    cj.wait()
    c1 = pltpu.async_copy(eu_h.at[idx_u], eu_v, gsem)
    c2 = pltpu.async_copy(ei_h.at[idx_i], ei_v, gsem)
    c3 = pltpu.async_copy(ub_h.at[idx_u], ub_v, gsem)
    c4 = pltpu.async_copy(ib_h.at[idx_i], ib_v, gsem)
    cb = pltpu.async_copy(b_h.at[z_idx], b_v, sem)
    c1.wait()
    c2.wait()

    lane = lax.iota(jnp.int32, L)
    zero16 = jnp.zeros((L,), jnp.int32)
    for r in range(R):
        acc = eu_v[r, pl.ds(0, L)] * ei_v[r, pl.ds(0, L)]
        for j in range(1, F // L):
            acc = acc + eu_v[r, pl.ds(j * L, L)] * ei_v[r, pl.ds(j * L, L)]
        acc_t[r, :] = acc
    s_vec = plsc.load_gather(acc_t, [lane, zero16])
    for j in range(1, L):
        s_vec = s_vec + plsc.load_gather(
            acc_t, [lane, jnp.full((L,), j, jnp.int32)])
    cb.wait()
    s_loc[...] = s_vec + b_v[...]

    pltpu.sync_copy(s_loc, s_sh.at[pl.ds(base, R)])
    c3.wait()
    c4.wait()
    rb16 = ub_v[...] + ib_v[...]
    plsc.subcore_barrier()
    pltpu.sync_copy(s_sh, s_full)

    s_reg = [s_full[pl.ds(j * L, L)] for j in range(B // L)]

    @pl.when(cid == 0)
    def _assemble_lo():
        for r in range(RW):
            rb = rb16[r]
            for j in range(F // L):
                out_loc[pl.ds(r * B + j * L, L)] = (
                    eu_v[r, pl.ds(j * L, L)] + (s_reg[j] + rb))
                out_loc[pl.ds(r * B + F + j * L, L)] = (
                    ei_v[r, pl.ds(j * L, L)] + (s_reg[F // L + j] + rb))

    @pl.when(cid == 1)
    def _assemble_hi():
        for r in range(RW):
            rb = rb16[RW + r]
            for j in range(F // L):
                out_loc[pl.ds(r * B + j * L, L)] = (
                    eu_v[RW + r, pl.ds(j * L, L)] + (s_reg[j] + rb))
                out_loc[pl.ds(r * B + F + j * L, L)] = (
                    ei_v[RW + r, pl.ds(j * L, L)] + (s_reg[F // L + j] + rb))

    pltpu.sync_copy(out_loc, out_h.at[pl.ds((base + RW * cid) * B, RW * B)])


def kernel(user, item, embed_user, embed_item, u_bias, i_bias, bias_):
    mesh = plsc.VectorSubcoreMesh(core_axis_name="c", subcore_axis_name="s")
    run = functools.partial(
        pl.kernel,
        mesh=mesh,
        compiler_params=pltpu.CompilerParams(
            needs_layout_passes=False,
            disable_bounds_checks=True,
            disable_semaphore_checks=True,
        ),
        out_type=jax.ShapeDtypeStruct((B * B,), jnp.float32),
        scratch_types=[
            pltpu.VMEM((R,), jnp.int32),
            pltpu.VMEM((R,), jnp.int32),
            pltpu.VMEM((L,), jnp.int32),
            pltpu.VMEM((R, F), jnp.float32),
            pltpu.VMEM((R, F), jnp.float32),
            pltpu.VMEM((R,), jnp.float32),
            pltpu.VMEM((R,), jnp.float32),
            pltpu.VMEM((L,), jnp.float32),
            pltpu.VMEM((R,), jnp.float32),
            pltpu.VMEM((R, L), jnp.float32),
            pltpu.VMEM((B,), jnp.float32),
            pltpu.VMEM((RW * B,), jnp.float32),
            pltpu.VMEM_SHARED((B,), jnp.float32),
            pltpu.SemaphoreType.DMA,
            pltpu.SemaphoreType.DMA,
        ],
    )(_body)
    return run(user, item, embed_user, embed_item,
               u_bias.reshape(-1), i_bias.reshape(-1), bias_)
